# baseline (speedup 1.0000x reference)
import jax
import jax.numpy as jnp
from jax import lax
from jax.experimental import pallas as pl
from jax.experimental.pallas import tpu as pltpu
from jax.experimental.pallas import tpu_sc as plsc

BATCH = 32
POINT_NUM = 6890
MAX_NB = 8
LANES = 16
NB_SLOTS = MAX_NB - 1
PADDED = 6896
GROUPS = PADDED // LANES
INV_N = 1.0 / (BATCH * POINT_NUM * 3)


def _sc_body(gt_hbm, pr_hbm, nbt_hbm, nn_hbm, out_hbm,
             d_v, t_v, nbt_v, nn_v, o_v, sem_a, sem_b):
    b = lax.axis_index("s") * 2 + lax.axis_index("c")

    cps = [
        pltpu.make_async_copy(gt_hbm.at[b], d_v, sem_a),
        pltpu.make_async_copy(pr_hbm.at[b], t_v, sem_a),
    ]
    cpn = [
        pltpu.make_async_copy(nbt_hbm, nbt_v, sem_b),
        pltpu.make_async_copy(nn_hbm, nn_v, sem_b),
    ]
    for c in cps:
        c.start()
    for c in cpn:
        c.start()
    for c in cps:
        c.wait()

    @plsc.parallel_loop(0, PADDED, LANES, unroll=4)
    def sub_body(o):
        s = pl.ds(o, LANES)
        d_v[0, s] = d_v[0, s] - t_v[0, s]
        d_v[1, s] = d_v[1, s] - t_v[1, s]
        d_v[2, s] = d_v[2, s] - t_v[2, s]

    for c in cpn:
        c.wait()

    rows = [jnp.full((LANES,), c, jnp.int32) for c in range(3)]

    def main_body(o, carry):
        lap, geo = carry
        s = pl.ds(o, LANES)
        nnv = nn_v[s]
        x = d_v[0, s]
        y = d_v[1, s]
        z = d_v[2, s]
        geo = geo + jnp.abs(x) + jnp.abs(y) + jnp.abs(z)
        idx = [nbt_v[n, s] for n in range(NB_SLOTS)]
        gx = [plsc.load_gather(d_v, [rows[0], i]) for i in idx]
        gy = [plsc.load_gather(d_v, [rows[1], i]) for i in idx]
        gz = [plsc.load_gather(d_v, [rows[2], i]) for i in idx]

        def tree7(g):
            return ((g[0] + g[1]) + (g[2] + g[3])) + ((g[4] + g[5]) + g[6])

        ax = x * nnv - tree7(gx)
        ay = y * nnv - tree7(gy)
        az = z * nnv - tree7(gz)
        lap = lap + jnp.abs(ax) + jnp.abs(ay) + jnp.abs(az)
        return lap, geo

    zero = jnp.zeros((LANES,), jnp.float32)
    lap, geo = plsc.parallel_loop(
        0, PADDED, LANES, unroll=4, carry=(zero, zero))(main_body)
    o_v[...] = (lap + geo) * INV_N
    pltpu.sync_copy(o_v, out_hbm.at[pl.ds(b * LANES, LANES)])


@jax.jit
def _mcloss(gt_t, pr_t, nbt, nn_p):
    call = pl.kernel(
        _sc_body,
        out_type=jax.ShapeDtypeStruct((BATCH * LANES,), jnp.float32),
        mesh=plsc.VectorSubcoreMesh(
            core_axis_name="c", subcore_axis_name="s",
            num_cores=2, num_subcores=16),
        compiler_params=pltpu.CompilerParams(needs_layout_passes=False),
        scratch_types=[
            pltpu.VMEM((3, PADDED), jnp.float32),
            pltpu.VMEM((3, PADDED), jnp.float32),
            pltpu.VMEM((NB_SLOTS, PADDED), jnp.int32),
            pltpu.VMEM((PADDED,), jnp.float32),
            pltpu.VMEM((LANES,), jnp.float32),
            pltpu.SemaphoreType.DMA,
            pltpu.SemaphoreType.DMA,
        ],
    )
    parts = call(gt_t, pr_t, nbt, nn_p)
    return jnp.sum(parts)


def kernel(gt_pc, predict_pc, neighbor_id_lstlst, neighbor_num_lst):
    pad = PADDED - POINT_NUM
    gt_t = jnp.pad(jnp.transpose(gt_pc, (0, 2, 1)), ((0, 0), (0, 0), (0, pad)))
    pr_t = jnp.pad(jnp.transpose(predict_pc, (0, 2, 1)),
                   ((0, 0), (0, 0), (0, pad)))
    nbt = jnp.pad(jnp.transpose(neighbor_id_lstlst[:, 1:], (1, 0)),
                  ((0, 0), (0, pad)), constant_values=POINT_NUM)
    nn_p = jnp.pad(neighbor_num_lst, (0, pad))
    return _mcloss(gt_t, pr_t, nbt, nn_p)


# speedup vs baseline: 1.1376x; 1.1376x over previous
import jax
import jax.numpy as jnp
from jax import lax
from jax.experimental import pallas as pl
from jax.experimental.pallas import tpu as pltpu
from jax.experimental.pallas import tpu_sc as plsc

BATCH = 32
POINT_NUM = 6890
MAX_NB = 8
LANES = 16
NB_SLOTS = MAX_NB - 1
PADDED = 6896
INV_N = 1.0 / (BATCH * POINT_NUM * 3)


def _sc_body(gt_hbm, pr_hbm, nbt_hbm, nn_hbm, out_hbm,
             d_v, t_v, nbt_v, nn_v, o_v, sem_a, sem_b):
    b = lax.axis_index("s") * 2 + lax.axis_index("c")

    cps = [
        pltpu.make_async_copy(gt_hbm.at[b], d_v, sem_a),
        pltpu.make_async_copy(pr_hbm.at[b], t_v, sem_a),
    ]
    cpn = [
        pltpu.make_async_copy(nbt_hbm, nbt_v, sem_b),
        pltpu.make_async_copy(nn_hbm, nn_v, sem_b),
    ]
    for c in cps:
        c.start()
    for c in cpn:
        c.start()
    for c in cps:
        c.wait()

    nfull = (POINT_NUM // LANES) * LANES

    @plsc.parallel_loop(0, nfull, LANES, unroll=4)
    def sub_body(o):
        s = pl.ds(o, LANES)
        d_v[0, s] = d_v[0, s] - t_v[0, s]
        d_v[1, s] = d_v[1, s] - t_v[1, s]
        d_v[2, s] = d_v[2, s] - t_v[2, s]

    st = pl.ds(POINT_NUM - LANES, LANES)
    covered = LANES - (POINT_NUM - nfull)
    tmask = lax.iota(jnp.int32, LANES) >= covered
    for c in range(3):
        d_v[c, st] = d_v[c, st] - jnp.where(tmask, t_v[c, st], 0.0)
    for c in cpn:
        c.wait()

    rows = [jnp.full((LANES,), c, jnp.int32) for c in range(3)]
    limit = jnp.full((LANES,), POINT_NUM - 1, jnp.int32)

    def main_body(o, carry):
        lap, geo, vb = carry
        s = pl.ds(o, LANES)
        vmask = vb < POINT_NUM
        nnv = nn_v[s]
        x = d_v[0, s]
        y = d_v[1, s]
        z = d_v[2, s]
        gabs = jnp.abs(x) + jnp.abs(y) + jnp.abs(z)
        geo = geo + jnp.where(vmask, gabs, 0.0)
        idx = [nbt_v[n, s] for n in range(NB_SLOTS)]
        masks = [i < POINT_NUM for i in idx]
        sidx = [jnp.minimum(i, limit) for i in idx]
        gx = [plsc.load_gather(d_v, [rows[0], i], mask=m)
              for i, m in zip(sidx, masks)]
        gy = [plsc.load_gather(d_v, [rows[1], i], mask=m)
              for i, m in zip(sidx, masks)]
        gz = [plsc.load_gather(d_v, [rows[2], i], mask=m)
              for i, m in zip(sidx, masks)]

        def tree7(g):
            return ((g[0] + g[1]) + (g[2] + g[3])) + ((g[4] + g[5]) + g[6])

        ax = x * nnv - tree7(gx)
        ay = y * nnv - tree7(gy)
        az = z * nnv - tree7(gz)
        labs = jnp.abs(ax) + jnp.abs(ay) + jnp.abs(az)
        lap = lap + jnp.where(vmask, labs, 0.0)
        return lap, geo, vb + LANES

    zero = jnp.zeros((LANES,), jnp.float32)
    vb0 = lax.iota(jnp.int32, LANES)
    lap, geo, _ = plsc.parallel_loop(
        0, PADDED, LANES, unroll=2, carry=(zero, zero, vb0))(main_body)
    o_v[...] = (lap + geo) * INV_N
    pltpu.sync_copy(o_v, out_hbm.at[pl.ds(b * LANES, LANES)])


@jax.jit
def _mcloss(gt_t, pr_t, nbt, nn_p):
    call = pl.kernel(
        _sc_body,
        out_type=jax.ShapeDtypeStruct((BATCH * LANES,), jnp.float32),
        mesh=plsc.VectorSubcoreMesh(
            core_axis_name="c", subcore_axis_name="s",
            num_cores=2, num_subcores=16),
        compiler_params=pltpu.CompilerParams(needs_layout_passes=False),
        scratch_types=[
            pltpu.VMEM((3, POINT_NUM), jnp.float32),
            pltpu.VMEM((3, POINT_NUM), jnp.float32),
            pltpu.VMEM((NB_SLOTS, PADDED), jnp.int32),
            pltpu.VMEM((PADDED,), jnp.float32),
            pltpu.VMEM((LANES,), jnp.float32),
            pltpu.SemaphoreType.DMA,
            pltpu.SemaphoreType.DMA,
        ],
    )
    parts = call(gt_t, pr_t, nbt, nn_p)
    return jnp.sum(parts)


def kernel(gt_pc, predict_pc, neighbor_id_lstlst, neighbor_num_lst):
    pad = PADDED - POINT_NUM
    gt_t = jnp.transpose(gt_pc, (0, 2, 1))
    pr_t = jnp.transpose(predict_pc, (0, 2, 1))
    nbt = jnp.pad(jnp.transpose(neighbor_id_lstlst[:, 1:], (1, 0)),
                  ((0, 0), (0, pad)), constant_values=POINT_NUM)
    nn_p = jnp.pad(neighbor_num_lst, (0, pad))
    return _mcloss(gt_t, pr_t, nbt, nn_p)


# speedup vs baseline: 1.1930x; 1.0486x over previous
import jax
import jax.numpy as jnp
from jax import lax
from jax.experimental import pallas as pl
from jax.experimental.pallas import tpu as pltpu
from jax.experimental.pallas import tpu_sc as plsc

BATCH = 32
POINT_NUM = 6890
MAX_NB = 8
LANES = 16
NB_SLOTS = MAX_NB - 1
PADDED = 6896
INV_N = 1.0 / (BATCH * POINT_NUM * 3)


def _sc_body(gt_hbm, pr_hbm, nbt_hbm, nn_hbm, out_hbm,
             d_v, t_v, nbt_v, nn_v, o_v, sem_a, sem_b):
    b = lax.axis_index("s") * 2 + lax.axis_index("c")

    cps = [
        pltpu.make_async_copy(gt_hbm.at[b], d_v, sem_a),
        pltpu.make_async_copy(pr_hbm.at[b], t_v, sem_a),
    ]
    cpn = [
        pltpu.make_async_copy(nbt_hbm, nbt_v, sem_b),
        pltpu.make_async_copy(nn_hbm, nn_v, sem_b),
    ]
    for c in cps:
        c.start()
    for c in cpn:
        c.start()
    for c in cps:
        c.wait()

    @plsc.parallel_loop(0, PADDED, LANES, unroll=4)
    def sub_body(o):
        s = pl.ds(o, LANES)
        d_v[0, s] = d_v[0, s] - t_v[0, s]
        d_v[1, s] = d_v[1, s] - t_v[1, s]
        d_v[2, s] = d_v[2, s] - t_v[2, s]

    for c in cpn:
        c.wait()

    rows = [jnp.full((LANES,), c, jnp.int32) for c in range(3)]

    def main_body(o, carry):
        lap, geo = carry
        s = pl.ds(o, LANES)
        nnv = nn_v[s]
        x = d_v[0, s]
        y = d_v[1, s]
        z = d_v[2, s]
        geo = geo + jnp.abs(x) + jnp.abs(y) + jnp.abs(z)
        idx = [nbt_v[n, s] for n in range(NB_SLOTS)]
        gx = [plsc.load_gather(d_v, [rows[0], i]) for i in idx]
        gy = [plsc.load_gather(d_v, [rows[1], i]) for i in idx]
        gz = [plsc.load_gather(d_v, [rows[2], i]) for i in idx]

        def tree7(g):
            return ((g[0] + g[1]) + (g[2] + g[3])) + ((g[4] + g[5]) + g[6])

        ax = x * nnv - tree7(gx)
        ay = y * nnv - tree7(gy)
        az = z * nnv - tree7(gz)
        lap = lap + jnp.abs(ax) + jnp.abs(ay) + jnp.abs(az)
        return lap, geo

    zero = jnp.zeros((LANES,), jnp.float32)
    lap, geo = plsc.parallel_loop(
        0, PADDED, LANES, unroll=2, carry=(zero, zero))(main_body)
    o_v[...] = (lap + geo) * INV_N
    pltpu.sync_copy(o_v, out_hbm.at[pl.ds(b * LANES, LANES)])


@jax.jit
def _mcloss(gt_t, pr_t, nbt, nn_p):
    call = pl.kernel(
        _sc_body,
        out_type=jax.ShapeDtypeStruct((BATCH * LANES,), jnp.float32),
        mesh=plsc.VectorSubcoreMesh(
            core_axis_name="c", subcore_axis_name="s",
            num_cores=2, num_subcores=16),
        compiler_params=pltpu.CompilerParams(needs_layout_passes=False),
        scratch_types=[
            pltpu.VMEM((3, PADDED), jnp.float32),
            pltpu.VMEM((3, PADDED), jnp.float32),
            pltpu.VMEM((NB_SLOTS, PADDED), jnp.int32),
            pltpu.VMEM((PADDED,), jnp.float32),
            pltpu.VMEM((LANES,), jnp.float32),
            pltpu.SemaphoreType.DMA,
            pltpu.SemaphoreType.DMA,
        ],
    )
    parts = call(gt_t, pr_t, nbt, nn_p)
    return jnp.sum(parts)


def kernel(gt_pc, predict_pc, neighbor_id_lstlst, neighbor_num_lst):
    pad = PADDED - POINT_NUM
    gt_t = jnp.pad(jnp.transpose(gt_pc, (0, 2, 1)), ((0, 0), (0, 0), (0, pad)))
    pr_t = jnp.pad(jnp.transpose(predict_pc, (0, 2, 1)),
                   ((0, 0), (0, 0), (0, pad)))
    nbt = jnp.pad(jnp.transpose(neighbor_id_lstlst[:, 1:], (1, 0)),
                  ((0, 0), (0, pad)), constant_values=POINT_NUM)
    nn_p = jnp.pad(neighbor_num_lst, (0, pad))
    return _mcloss(gt_t, pr_t, nbt, nn_p)


# speedup vs baseline: 1.1942x; 1.0011x over previous
import jax
import jax.numpy as jnp
from jax import lax
from jax.experimental import pallas as pl
from jax.experimental.pallas import tpu as pltpu
from jax.experimental.pallas import tpu_sc as plsc

BATCH = 32
POINT_NUM = 6890
MAX_NB = 8
LANES = 16
NB_SLOTS = MAX_NB - 1
PADDED = 6896
INV_N = 1.0 / (BATCH * POINT_NUM * 3)


def _sc_body(gt_hbm, pr_hbm, nbt_hbm, nn_hbm, out_hbm,
             d_v, t_v, nbt_v, nn_v, o_v, sem_a, sem_b):
    b = lax.axis_index("s") * 2 + lax.axis_index("c")

    cps = [
        pltpu.make_async_copy(gt_hbm.at[b], d_v, sem_a),
        pltpu.make_async_copy(pr_hbm.at[b], t_v, sem_a),
    ]
    cpn = [
        pltpu.make_async_copy(nbt_hbm, nbt_v, sem_b),
        pltpu.make_async_copy(nn_hbm, nn_v, sem_b),
    ]
    for c in cps:
        c.start()
    for c in cpn:
        c.start()
    for c in cps:
        c.wait()

    @plsc.parallel_loop(0, PADDED, LANES, unroll=4)
    def sub_body(o):
        s = pl.ds(o, LANES)
        d_v[0, s] = d_v[0, s] - t_v[0, s]
        d_v[1, s] = d_v[1, s] - t_v[1, s]
        d_v[2, s] = d_v[2, s] - t_v[2, s]

    for c in cpn:
        c.wait()

    rows = [jnp.full((LANES,), c, jnp.int32) for c in range(3)]

    def main_body(o, carry):
        lap, geo = carry
        s = pl.ds(o, LANES)
        nnv = nn_v[s]
        x = d_v[0, s]
        y = d_v[1, s]
        z = d_v[2, s]
        geo = geo + jnp.abs(x) + jnp.abs(y) + jnp.abs(z)
        idx = [nbt_v[n, s] for n in range(NB_SLOTS)]
        gx = [plsc.load_gather(d_v, [rows[0], i]) for i in idx]
        gy = [plsc.load_gather(d_v, [rows[1], i]) for i in idx]
        gz = [plsc.load_gather(d_v, [rows[2], i]) for i in idx]

        def tree7(g):
            return ((g[0] + g[1]) + (g[2] + g[3])) + ((g[4] + g[5]) + g[6])

        ax = x * nnv - tree7(gx)
        ay = y * nnv - tree7(gy)
        az = z * nnv - tree7(gz)
        lap = lap + jnp.abs(ax) + jnp.abs(ay) + jnp.abs(az)
        return lap, geo

    zero = jnp.zeros((LANES,), jnp.float32)
    lap, geo = plsc.parallel_loop(
        0, PADDED, LANES, unroll=2, carry=(zero, zero))(main_body)
    o_v[...] = (lap + geo) * INV_N
    pltpu.sync_copy(o_v, out_hbm.at[pl.ds(b * LANES, LANES)])


@jax.jit
def _mcloss(gt_t, pr_t, nbt, nn_p):
    call = pl.kernel(
        _sc_body,
        out_type=jax.ShapeDtypeStruct((BATCH * LANES,), jnp.float32),
        mesh=plsc.VectorSubcoreMesh(
            core_axis_name="c", subcore_axis_name="s",
            num_cores=2, num_subcores=16),
        compiler_params=pltpu.CompilerParams(needs_layout_passes=False),
        scratch_types=[
            pltpu.VMEM((3, PADDED), jnp.float32),
            pltpu.VMEM((3, PADDED), jnp.float32),
            pltpu.VMEM((NB_SLOTS, PADDED), jnp.int32),
            pltpu.VMEM((PADDED,), jnp.float32),
            pltpu.VMEM((LANES,), jnp.float32),
            pltpu.SemaphoreType.DMA,
            pltpu.SemaphoreType.DMA,
        ],
    )
    parts = call(gt_t, pr_t, nbt, nn_p)
    return jnp.sum(parts)


def kernel(gt_pc, predict_pc, neighbor_id_lstlst, neighbor_num_lst):
    pad = PADDED - POINT_NUM
    zeros3 = jnp.zeros((BATCH, 3, PADDED), jnp.float32)
    gt_t = lax.dynamic_update_slice(zeros3, jnp.transpose(gt_pc, (0, 2, 1)),
                                    (0, 0, 0))
    pr_t = lax.dynamic_update_slice(zeros3,
                                    jnp.transpose(predict_pc, (0, 2, 1)),
                                    (0, 0, 0))
    nbt = jnp.pad(jnp.transpose(neighbor_id_lstlst[:, 1:], (1, 0)),
                  ((0, 0), (0, pad)), constant_values=POINT_NUM)
    nn_p = jnp.pad(neighbor_num_lst, (0, pad))
    return _mcloss(gt_t, pr_t, nbt, nn_p)
